# Initial kernel scaffold; baseline (speedup 1.0000x reference)
import jax
import jax.numpy as jnp
from jax.experimental import pallas as pl


def kernel(input_vect, weights, epoch):
    raise NotImplementedError("write your pallas kernel here")


# speedup vs baseline: 13.8124x; 13.8124x over previous
import jax
import jax.numpy as jnp
from jax.experimental import pallas as pl
from jax.experimental.pallas import tpu as pltpu

_B = 512
_K = 1024
_D = 256
_GRID = 32

_MAX_EPOCHS = 100
_INITIAL_RADIUS = 16.0
_INITIAL_LR = 0.1
_STD_COEFF = 0.5


def _som_kernel(x_ref, xa_ref, wt_ref, scal_ref, idx_ref, nw_ref):
    x = x_ref[...]
    wt = wt_ref[...]
    alpha = scal_ref[0]
    neg_inv_two_sigma_sq = scal_ref[1]

    wsq = jnp.sum(wt * wt, axis=0, keepdims=True)
    xw = jax.lax.dot_general(
        x, wt, (((1,), (0,)), ((), ())),
        preferred_element_type=jnp.float32,
        precision=jax.lax.Precision.HIGHEST,
    )
    d = wsq - 2.0 * xw

    m = jnp.min(d, axis=1, keepdims=True)
    kio = jax.lax.broadcasted_iota(jnp.int32, (_B, _K), 1)
    idx = jnp.min(jnp.where(d == m, kio, jnp.int32(_K)), axis=1,
                  keepdims=True)
    idx_ref[...] = idx

    idx_t = idx.reshape(1, _B)
    bi = idx_t >> 5
    bj = idx_t & (_GRID - 1)
    kio_t = jax.lax.broadcasted_iota(jnp.int32, (_K, _B), 0)
    li = kio_t >> 5
    lj = kio_t & (_GRID - 1)
    d2 = (li - bi) * (li - bi) + (lj - bj) * (lj - bj)
    lr_t = alpha * jnp.exp(d2.astype(jnp.float32) * neg_inv_two_sigma_sq)

    xa = xa_ref[...]
    num = jax.lax.dot_general(
        lr_t, xa, (((1,), (0,)), ((), ())),
        preferred_element_type=jnp.float32,
        precision=jax.lax.Precision.HIGHEST,
    )
    den = num[:, _D:_D + 1] + 1e-12
    nw_ref[...] = num[:, :_D] / den


def _som(input_vect, weights, alpha, neg_inv_two_sigma_sq):
    pad = jnp.zeros((_B, 128), dtype=jnp.float32).at[:, 0].set(1.0)
    xa = jnp.concatenate([input_vect, pad], axis=1)
    wt = weights.T
    scal = jnp.stack([alpha, neg_inv_two_sigma_sq])
    idx, new_w = pl.pallas_call(
        _som_kernel,
        out_shape=[
            jax.ShapeDtypeStruct((_B, 1), jnp.int32),
            jax.ShapeDtypeStruct((_K, _D), jnp.float32),
        ],
        in_specs=[
            pl.BlockSpec(memory_space=pltpu.VMEM),
            pl.BlockSpec(memory_space=pltpu.VMEM),
            pl.BlockSpec(memory_space=pltpu.VMEM),
            pl.BlockSpec(memory_space=pltpu.SMEM),
        ],
        out_specs=[
            pl.BlockSpec(memory_space=pltpu.VMEM),
            pl.BlockSpec(memory_space=pltpu.VMEM),
        ],
    )(input_vect, xa, wt, scal)
    idx = idx[:, 0]
    bmu_locs = jnp.stack([idx >> 5, idx & (_GRID - 1)], axis=1)
    return bmu_locs, new_w


def kernel(input_vect, weights, epoch):
    epoch_f = jnp.asarray(epoch, dtype=jnp.float32)
    radius = _INITIAL_RADIUS - epoch_f * (
        (_INITIAL_RADIUS - 1.0) / float(_MAX_EPOCHS - 1))
    alpha = _INITIAL_LR * (1.0 - epoch_f / float(_MAX_EPOCHS))
    sigma = radius * _STD_COEFF
    neg_inv_two_sigma_sq = -1.0 / (2.0 * sigma * sigma)
    return _som(input_vect, weights, alpha, neg_inv_two_sigma_sq)


# speedup vs baseline: 19.4686x; 1.4095x over previous
import jax
import jax.numpy as jnp
from jax.experimental import pallas as pl
from jax.experimental.pallas import tpu as pltpu

_B = 512
_K = 1024
_D = 256
_GRID = 32

_MAX_EPOCHS = 100
_INITIAL_RADIUS = 16.0
_INITIAL_LR = 0.1
_STD_COEFF = 0.5


def _som_kernel(x_ref, wt_ref, scal_ref, idx_ref, nw_ref):
    x = x_ref[...]
    wt = wt_ref[...]
    alpha = scal_ref[0]
    neg_inv_two_sigma_sq = scal_ref[1]

    wsq = jnp.sum(wt * wt, axis=0, keepdims=True)
    xw = jax.lax.dot_general(
        x, wt, (((1,), (0,)), ((), ())),
        preferred_element_type=jnp.float32,
        precision=jax.lax.Precision.HIGHEST,
    )
    d = wsq - 2.0 * xw

    m = jnp.min(d, axis=1, keepdims=True)
    kio = jax.lax.broadcasted_iota(jnp.int32, (_B, _K), 1)
    idx = jnp.min(jnp.where(d == m, kio, jnp.int32(_K)), axis=1,
                  keepdims=True)
    idx_ref[...] = idx

    idx_t = idx.reshape(1, _B)
    bi = idx_t >> 5
    bj = idx_t & (_GRID - 1)
    kio_t = jax.lax.broadcasted_iota(jnp.int32, (_K, _B), 0)
    li = kio_t >> 5
    lj = kio_t & (_GRID - 1)
    d2 = (li - bi) * (li - bi) + (lj - bj) * (lj - bj)
    lr_t = alpha * jnp.exp(d2.astype(jnp.float32) * neg_inv_two_sigma_sq)

    den = jnp.sum(lr_t, axis=1, keepdims=True) + 1e-12
    num = jax.lax.dot_general(
        lr_t.astype(jnp.bfloat16), x.astype(jnp.bfloat16),
        (((1,), (0,)), ((), ())),
        preferred_element_type=jnp.float32,
    )
    nw_ref[...] = num / den


def _som(input_vect, weights, alpha, neg_inv_two_sigma_sq):
    wt = weights.T
    scal = jnp.stack([alpha, neg_inv_two_sigma_sq])
    idx, new_w = pl.pallas_call(
        _som_kernel,
        out_shape=[
            jax.ShapeDtypeStruct((_B, 1), jnp.int32),
            jax.ShapeDtypeStruct((_K, _D), jnp.float32),
        ],
        in_specs=[
            pl.BlockSpec(memory_space=pltpu.VMEM),
            pl.BlockSpec(memory_space=pltpu.VMEM),
            pl.BlockSpec(memory_space=pltpu.SMEM),
        ],
        out_specs=[
            pl.BlockSpec(memory_space=pltpu.VMEM),
            pl.BlockSpec(memory_space=pltpu.VMEM),
        ],
    )(input_vect, wt, scal)
    idx = idx[:, 0]
    bmu_locs = jnp.stack([idx >> 5, idx & (_GRID - 1)], axis=1)
    return bmu_locs, new_w


def kernel(input_vect, weights, epoch):
    epoch_f = jnp.asarray(epoch, dtype=jnp.float32)
    radius = _INITIAL_RADIUS - epoch_f * (
        (_INITIAL_RADIUS - 1.0) / float(_MAX_EPOCHS - 1))
    alpha = _INITIAL_LR * (1.0 - epoch_f / float(_MAX_EPOCHS))
    sigma = radius * _STD_COEFF
    neg_inv_two_sigma_sq = -1.0 / (2.0 * sigma * sigma)
    return _som(input_vect, weights, alpha, neg_inv_two_sigma_sq)


# speedup vs baseline: 21.7530x; 1.1173x over previous
import jax
import jax.numpy as jnp
from jax.experimental import pallas as pl
from jax.experimental.pallas import tpu as pltpu

_B = 512
_K = 1024
_D = 256
_GRID = 32

_MAX_EPOCHS = 100
_INITIAL_RADIUS = 16.0
_INITIAL_LR = 0.1
_STD_COEFF = 0.5


def _som_kernel(x_ref, xt_ref, w_ref, epoch_ref, locs_ref, nw_ref):
    x = x_ref[...]
    xt = xt_ref[...]
    w = w_ref[...]

    epoch_f = epoch_ref[0]
    radius = _INITIAL_RADIUS - epoch_f * (
        (_INITIAL_RADIUS - 1.0) / float(_MAX_EPOCHS - 1))
    alpha = _INITIAL_LR * (1.0 - epoch_f / float(_MAX_EPOCHS))
    sigma = radius * _STD_COEFF
    neg_inv_two_sigma_sq = -1.0 / (2.0 * sigma * sigma)

    wsq = jnp.sum(w * w, axis=1, keepdims=True)
    wx = jax.lax.dot_general(
        w, xt, (((1,), (0,)), ((), ())),
        preferred_element_type=jnp.float32,
        precision=jax.lax.Precision.HIGHEST,
    )
    d = wsq - 2.0 * wx

    m = jnp.min(d, axis=0, keepdims=True)
    kio_t = jax.lax.broadcasted_iota(jnp.int32, (_K, _B), 0)
    idx_t = jnp.min(jnp.where(d == m, kio_t, jnp.int32(_K)), axis=0,
                    keepdims=True)
    idx = idx_t.reshape(_B, 1)
    locs_ref[...] = jnp.concatenate([idx >> 5, idx & (_GRID - 1)], axis=1)

    bi = idx_t >> 5
    bj = idx_t & (_GRID - 1)
    li = kio_t >> 5
    lj = kio_t & (_GRID - 1)
    d2 = (li - bi) * (li - bi) + (lj - bj) * (lj - bj)
    lr_t = alpha * jnp.exp(d2.astype(jnp.float32) * neg_inv_two_sigma_sq)

    den = jnp.sum(lr_t, axis=1, keepdims=True) + 1e-12
    num = jax.lax.dot_general(
        lr_t.astype(jnp.bfloat16), x.astype(jnp.bfloat16),
        (((1,), (0,)), ((), ())),
        preferred_element_type=jnp.float32,
    )
    nw_ref[...] = num / den


def kernel(input_vect, weights, epoch):
    epoch_f = jnp.asarray(epoch, dtype=jnp.float32).reshape(1)
    xt = input_vect.T
    return pl.pallas_call(
        _som_kernel,
        out_shape=[
            jax.ShapeDtypeStruct((_B, 2), jnp.int32),
            jax.ShapeDtypeStruct((_K, _D), jnp.float32),
        ],
        in_specs=[
            pl.BlockSpec(memory_space=pltpu.VMEM),
            pl.BlockSpec(memory_space=pltpu.VMEM),
            pl.BlockSpec(memory_space=pltpu.VMEM),
            pl.BlockSpec(memory_space=pltpu.SMEM),
        ],
        out_specs=[
            pl.BlockSpec(memory_space=pltpu.VMEM),
            pl.BlockSpec(memory_space=pltpu.VMEM),
        ],
    )(input_vect, xt, weights, epoch_f)


# speedup vs baseline: 27.0545x; 1.2437x over previous
import jax
import jax.numpy as jnp
from jax.experimental import pallas as pl
from jax.experimental.pallas import tpu as pltpu

_B = 512
_K = 1024
_D = 256
_GRID = 32

_MAX_EPOCHS = 100
_INITIAL_RADIUS = 16.0
_INITIAL_LR = 0.1
_STD_COEFF = 0.5


def _som_kernel(x_ref, w_ref, epoch_ref, locs_ref, nw_ref, xt_ref):
    x = x_ref[...]
    w = w_ref[...]
    xt_ref[...] = x.T
    xt = xt_ref[...]

    epoch_f = epoch_ref[0]
    radius = _INITIAL_RADIUS - epoch_f * (
        (_INITIAL_RADIUS - 1.0) / float(_MAX_EPOCHS - 1))
    alpha = _INITIAL_LR * (1.0 - epoch_f / float(_MAX_EPOCHS))
    sigma = radius * _STD_COEFF
    neg_inv_two_sigma_sq = -1.0 / (2.0 * sigma * sigma)

    wsq = jnp.sum(w * w, axis=1, keepdims=True)
    wx = jax.lax.dot_general(
        w, xt, (((1,), (0,)), ((), ())),
        preferred_element_type=jnp.float32,
        precision=jax.lax.Precision.HIGHEST,
    )
    d = wsq - 2.0 * wx

    m = jnp.min(d, axis=0, keepdims=True)
    kio_t = jax.lax.broadcasted_iota(jnp.int32, (_K, _B), 0)
    idx_t = jnp.min(jnp.where(d == m, kio_t, jnp.int32(_K)), axis=0,
                    keepdims=True)
    idx = idx_t.reshape(_B, 1)
    locs_ref[...] = jnp.concatenate([idx >> 5, idx & (_GRID - 1)], axis=1)

    bi = idx_t >> 5
    bj = idx_t & (_GRID - 1)
    li = kio_t >> 5
    lj = kio_t & (_GRID - 1)
    d2 = (li - bi) * (li - bi) + (lj - bj) * (lj - bj)
    lr_t = alpha * jnp.exp(d2.astype(jnp.float32) * neg_inv_two_sigma_sq)

    den = jnp.sum(lr_t, axis=1, keepdims=True) + 1e-12
    num = jax.lax.dot_general(
        lr_t.astype(jnp.bfloat16), x.astype(jnp.bfloat16),
        (((1,), (0,)), ((), ())),
        preferred_element_type=jnp.float32,
    )
    nw_ref[...] = num / den


def kernel(input_vect, weights, epoch):
    epoch_f = jnp.asarray(epoch, dtype=jnp.float32).reshape(1)
    return pl.pallas_call(
        _som_kernel,
        out_shape=[
            jax.ShapeDtypeStruct((_B, 2), jnp.int32),
            jax.ShapeDtypeStruct((_K, _D), jnp.float32),
        ],
        in_specs=[
            pl.BlockSpec(memory_space=pltpu.VMEM),
            pl.BlockSpec(memory_space=pltpu.VMEM),
            pl.BlockSpec(memory_space=pltpu.SMEM),
        ],
        out_specs=[
            pl.BlockSpec(memory_space=pltpu.VMEM),
            pl.BlockSpec(memory_space=pltpu.VMEM),
        ],
        scratch_shapes=[pltpu.VMEM((_D, _B), jnp.float32)],
    )(input_vect, weights, epoch_f)
